# baseline (speedup 1.0000x reference)
import functools

import jax
import jax.numpy as jnp
from jax import lax
from jax.experimental import pallas as pl
from jax.experimental.pallas import tpu as pltpu
from jax.experimental.pallas import tpu_sc as plsc

_NC = 2
_NS = 16
_NW = _NC * _NS
_CK = 64
_CPS = 4


def _make_sc_kernel(N, D, E_pad, N_pad):
    T = E_pad // _NS
    STEPS = T // (_CK * _CPS)
    RPT = N_pad // _NS
    mesh = plsc.VectorSubcoreMesh(core_axis_name="c", subcore_axis_name="s")

    @functools.partial(
        pl.kernel,
        mesh=mesh,
        compiler_params=pltpu.CompilerParams(needs_layout_passes=False),
        out_type=(
            jax.ShapeDtypeStruct((_NC * N_pad, D), jnp.float32),
            jax.ShapeDtypeStruct((_NW * N_pad,), jnp.float32),
        ),
        scratch_types=[
            pltpu.VMEM((_CPS, _CK), jnp.int32),
            pltpu.VMEM((_CPS, _CK), jnp.int32),
            pltpu.VMEM((_CPS * _CK, D), jnp.float32),
            pltpu.VMEM((N_pad,), jnp.float32),
            pltpu.VMEM((16, D), jnp.float32),
            pltpu.VMEM_SHARED((N_pad, D), jnp.float32),
            pltpu.SemaphoreType.DMA((_CPS,)),
            pltpu.SemaphoreType.DMA((_CPS,)),
            pltpu.SemaphoreType.DMA((_CPS,)),
        ],
    )
    def sc(x_hbm, src_hbm, dst_hbm, sums_hbm, cnts_hbm,
           idx_s, idx_d, rows, cnt_loc, zbuf, sums_sh,
           sem_i, sem_g, sem_s):
        c = lax.axis_index("c")
        s = lax.axis_index("s")
        w = c * _NS + s
        zv = jnp.zeros((16,), jnp.float32)
        ones = jnp.ones((16,), jnp.float32)

        def zb_body(i, _):
            zbuf[i // (D // 16), pl.ds((i % (D // 16)) * 16, 16)] = zv
            return 0
        lax.fori_loop(0, 16 * (D // 16), zb_body, 0)

        def zc_body(i, _):
            cnt_loc[pl.ds(i * 16, 16)] = zv
            return 0
        lax.fori_loop(0, N_pad // 16, zc_body, 0)

        zcps = [
            pltpu.async_copy(
                zbuf, sums_sh.at[pl.ds(s * RPT + i * 16, 16)], sem_g.at[0])
            for i in range(RPT // 16)
        ]
        for cp in zcps:
            cp.wait()
        plsc.subcore_barrier()

        base_row = s * (T // _CK)

        def step(i, _):
            r0 = base_row + i * _CPS
            icps = []
            for j in range(_CPS):
                icps.append(pltpu.async_copy(
                    src_hbm.at[r0 + j], idx_s.at[j], sem_i.at[j]))
                icps.append(pltpu.async_copy(
                    dst_hbm.at[r0 + j], idx_d.at[j], sem_i.at[j]))
            gcps = []
            for j in range(_CPS):
                icps[2 * j].wait()
                gcps.append(pltpu.async_copy(
                    x_hbm.at[idx_s.at[j]],
                    rows.at[pl.ds(j * _CK, _CK)], sem_g.at[j]))
            scps = []
            for j in range(_CPS):
                icps[2 * j + 1].wait()
                gcps[j].wait()
                scps.append(pltpu.async_copy(
                    rows.at[pl.ds(j * _CK, _CK)],
                    sums_sh.at[idx_d.at[j]], sem_s.at[j], add=True))
                for q in range(_CK // 16):
                    vidx = idx_d[j, pl.ds(q * 16, 16)]
                    plsc.addupdate_scatter(cnt_loc, [vidx], ones)
            for cp in scps:
                cp.wait()
            return 0
        lax.fori_loop(0, jnp.where(c == 0, STEPS, 0), step, 0)

        plsc.subcore_barrier()
        pltpu.sync_copy(
            sums_sh.at[pl.ds(s * RPT, RPT)],
            sums_hbm.at[pl.ds(c * N_pad + s * RPT, RPT)])
        pltpu.sync_copy(cnt_loc, cnts_hbm.at[pl.ds(w * N_pad, N_pad)])

    return sc


def _tc_body(sums_ref, cnts_ref, w_ref, x_ref, o_ref):
    p = sums_ref[0] + sums_ref[1]
    cnt = jnp.sum(cnts_ref[...], axis=0)
    mm = lax.dot_general(p, w_ref[...],
                         dimension_numbers=(((1,), (1,)), ((), ())),
                         preferred_element_type=jnp.float32)
    o_ref[...] = mm / jnp.maximum(cnt, 1.0)[:, None] + x_ref[...]


def kernel(x, edge_index, W_rel):
    N, D = x.shape
    E = edge_index.shape[1]
    _GG = _CK * _CPS
    T = -(-E // (_NS * _GG)) * _GG
    E_pad = T * _NS
    N_pad = -(-(N + 1) // (_NS * 16)) * (_NS * 16)

    src = edge_index[0]
    dst = edge_index[1]
    pad = E_pad - E
    if pad:
        src = jnp.concatenate([src, jnp.zeros((pad,), jnp.int32)])
        dst = jnp.concatenate([dst, jnp.full((pad,), N, jnp.int32)])
    src2 = src.reshape(E_pad // _CK, _CK)
    dst2 = dst.reshape(E_pad // _CK, _CK)

    sums, cnts = _make_sc_kernel(N, D, E_pad, N_pad)(x, src2, dst2)
    sums = sums.reshape(_NC, N_pad, D)
    cnts = cnts.reshape(_NW, N_pad)

    BR = 1024
    NB = N_pad // BR
    out = pl.pallas_call(
        _tc_body,
        grid=(NB,),
        in_specs=[
            pl.BlockSpec((_NC, BR, D), lambda i: (0, i, 0)),
            pl.BlockSpec((_NW, BR), lambda i: (0, i)),
            pl.BlockSpec((D, D), lambda i: (0, 0)),
            pl.BlockSpec((BR, D), lambda i: (i, 0)),
        ],
        out_specs=pl.BlockSpec((BR, D), lambda i: (i, 0)),
        out_shape=jax.ShapeDtypeStruct((N, D), jnp.float32),
    )(sums, cnts, W_rel, x)
    return out


# speedup vs baseline: 1.5116x; 1.5116x over previous
import functools

import jax
import jax.numpy as jnp
from jax import lax
from jax.experimental import pallas as pl
from jax.experimental.pallas import tpu as pltpu
from jax.experimental.pallas import tpu_sc as plsc

_NC = 2
_NS = 16
_NW = _NC * _NS
_CK = 64
_CPS = 4


def _make_sc_kernel(N, D, E0, E1, N_pad):
    T0 = E0 // _NS
    T1 = E1 // _NS
    RPT = N_pad // _NS
    mesh = plsc.VectorSubcoreMesh(core_axis_name="c", subcore_axis_name="s")

    @functools.partial(
        pl.kernel,
        mesh=mesh,
        compiler_params=pltpu.CompilerParams(needs_layout_passes=False),
        out_type=(
            jax.ShapeDtypeStruct((_NC * N_pad, D), jnp.float32),
            jax.ShapeDtypeStruct((_NW * N_pad,), jnp.float32),
        ),
        scratch_types=[
            pltpu.VMEM((_CPS, _CK), jnp.int32),
            pltpu.VMEM((_CPS, _CK), jnp.int32),
            pltpu.VMEM((_CPS * _CK, D), jnp.float32),
            pltpu.VMEM((N_pad,), jnp.float32),
            pltpu.VMEM((16, D), jnp.float32),
            pltpu.VMEM_SHARED((N_pad, D), jnp.float32),
            pltpu.SemaphoreType.DMA((_CPS,)),
            pltpu.SemaphoreType.DMA((_CPS,)),
            pltpu.SemaphoreType.DMA((_CPS,)),
        ],
    )
    def sc(x_hbm, src_hbm, dst_hbm, sums_hbm, cnts_hbm,
           idx_s, idx_d, rows, cnt_loc, zbuf, sums_sh,
           sem_i, sem_g, sem_s):
        c = lax.axis_index("c")
        s = lax.axis_index("s")
        w = c * _NS + s
        zv = jnp.zeros((16,), jnp.float32)
        ones = jnp.ones((16,), jnp.float32)

        def zb_body(i, _):
            zbuf[i // (D // 16), pl.ds((i % (D // 16)) * 16, 16)] = zv
            return 0
        lax.fori_loop(0, 16 * (D // 16), zb_body, 0)

        def zc_body(i, _):
            cnt_loc[pl.ds(i * 16, 16)] = zv
            return 0
        lax.fori_loop(0, N_pad // 16, zc_body, 0)

        zcps = [
            pltpu.async_copy(
                zbuf, sums_sh.at[pl.ds(s * RPT + i * 16, 16)], sem_g.at[0])
            for i in range(RPT // 16)
        ]
        for cp in zcps:
            cp.wait()
        plsc.subcore_barrier()

        base_row = jnp.where(c == 0, s * (T0 // _CK),
                             E0 // _CK + s * (T1 // _CK))

        def step(i, _):
            r0 = base_row + i * _CPS
            icps = []
            for j in range(_CPS):
                icps.append(pltpu.async_copy(
                    src_hbm.at[r0 + j], idx_s.at[j], sem_i.at[j]))
                icps.append(pltpu.async_copy(
                    dst_hbm.at[r0 + j], idx_d.at[j], sem_i.at[j]))
            gcps = []
            for j in range(_CPS):
                icps[2 * j].wait()
                gcps.append(pltpu.async_copy(
                    x_hbm.at[idx_s.at[j]],
                    rows.at[pl.ds(j * _CK, _CK)], sem_g.at[j]))
            scps = []
            for j in range(_CPS):
                icps[2 * j + 1].wait()
                gcps[j].wait()
                scps.append(pltpu.async_copy(
                    rows.at[pl.ds(j * _CK, _CK)],
                    sums_sh.at[idx_d.at[j]], sem_s.at[j], add=True))
                for q in range(_CK // 16):
                    vidx = idx_d[j, pl.ds(q * 16, 16)]
                    plsc.addupdate_scatter(cnt_loc, [vidx], ones)
            for cp in scps:
                cp.wait()
            return 0
        lax.fori_loop(
            0, jnp.where(c == 0, T0 // (_CK * _CPS), T1 // (_CK * _CPS)),
            step, 0)

        plsc.subcore_barrier()
        pltpu.sync_copy(
            sums_sh.at[pl.ds(s * RPT, RPT)],
            sums_hbm.at[pl.ds(c * N_pad + s * RPT, RPT)])
        pltpu.sync_copy(cnt_loc, cnts_hbm.at[pl.ds(w * N_pad, N_pad)])

    return sc


def _tc_body(sums_ref, cnts_ref, w_ref, x_ref, o_ref):
    p = sums_ref[0] + sums_ref[1]
    cnt = jnp.sum(cnts_ref[...], axis=0)
    mm = lax.dot_general(p, w_ref[...],
                         dimension_numbers=(((1,), (1,)), ((), ())),
                         preferred_element_type=jnp.float32)
    o_ref[...] = mm / jnp.maximum(cnt, 1.0)[:, None] + x_ref[...]


def kernel(x, edge_index, W_rel):
    N, D = x.shape
    E = edge_index.shape[1]
    _GG = _CK * _CPS
    _Q = _NS * _GG
    E0 = (int(E * 0.70) // _Q) * _Q
    E1 = -(-(E - E0) // _Q) * _Q
    E_pad = E0 + E1
    N_pad = -(-(N + 1) // (_NS * 16)) * (_NS * 16)

    src = edge_index[0]
    dst = edge_index[1]
    pad = E_pad - E
    if pad:
        src = jnp.concatenate([src, jnp.zeros((pad,), jnp.int32)])
        dst = jnp.concatenate([dst, jnp.full((pad,), N, jnp.int32)])
    src2 = src.reshape(E_pad // _CK, _CK)
    dst2 = dst.reshape(E_pad // _CK, _CK)

    sums, cnts = _make_sc_kernel(N, D, E0, E1, N_pad)(x, src2, dst2)
    sums = sums.reshape(_NC, N_pad, D)
    cnts = cnts.reshape(_NW, N_pad)

    BR = 1024
    NB = N_pad // BR
    out = pl.pallas_call(
        _tc_body,
        grid=(NB,),
        in_specs=[
            pl.BlockSpec((_NC, BR, D), lambda i: (0, i, 0)),
            pl.BlockSpec((_NW, BR), lambda i: (0, i)),
            pl.BlockSpec((D, D), lambda i: (0, 0)),
            pl.BlockSpec((BR, D), lambda i: (i, 0)),
        ],
        out_specs=pl.BlockSpec((BR, D), lambda i: (i, 0)),
        out_shape=jax.ShapeDtypeStruct((N, D), jnp.float32),
    )(sums, cnts, W_rel, x)
    return out
